# baseline (speedup 1.0000x reference)
import functools
import jax
import jax.numpy as jnp
from jax import lax
from jax.experimental import pallas as pl
from jax.experimental.pallas import tpu as pltpu

_N = 10000
_E = 320000
_C = 128
_EMB = 512
_NET = 7
_GROUP = 32
_CPG = _C // _GROUP
_BS = 8
_EPS = 1e-05
_EROWS = _E // 128


def _group_mat():
    i = jnp.arange(_C) // _CPG
    return (i[:, None] == i[None, :]).astype(jnp.float32)


def _gn_silu(data, bid_col, w, b, gmat):
    onehot = (bid_col == lax.broadcasted_iota(jnp.int32, (_N, _BS), 1)).astype(
        jnp.float32)
    dn = (((0,), (0,)), ((), ()))
    ones = jnp.ones((_N, 1), jnp.float32)
    cnt = lax.dot_general(onehot, ones, dn,
                          preferred_element_type=jnp.float32)
    inv_cnt = 1.0 / (cnt * _CPG + _EPS)
    sums = lax.dot_general(onehot, data, dn,
                           preferred_element_type=jnp.float32)
    mean = (sums * inv_cnt) @ gmat
    centered = data - jnp.dot(onehot, mean,
                              preferred_element_type=jnp.float32)
    var = lax.dot_general(onehot, centered * centered, dn,
                          preferred_element_type=jnp.float32) * inv_cnt
    inv_std = 1.0 / jnp.sqrt(var @ gmat + _EPS)
    out = centered * jnp.dot(onehot, inv_std,
                             preferred_element_type=jnp.float32)
    out = out * w + b
    return out * jax.nn.sigmoid(out)


def _gn1_body(x_ref, bid_ref, w_ref, b_ref, g_ref, o_ref):
    o_ref[...] = _gn_silu(x_ref[...], bid_ref[...], w_ref[...], b_ref[...],
                          g_ref[...])


def _gn2_body(h_ref, emb_ref, ew_ref, eb_ref, bid_ref, w_ref, b_ref, g_ref,
              o_ref):
    e = emb_ref[...]
    e = e * jax.nn.sigmoid(e)
    emb_out = lax.dot_general(e, ew_ref[...], (((1,), (1,)), ((), ())),
                              preferred_element_type=jnp.float32)
    data = h_ref[...] + emb_out + eb_ref[...]
    o_ref[...] = _gn_silu(data, bid_ref[...], w_ref[...], b_ref[...],
                          g_ref[...])


def _counts_body(flat_ref, inv_ref):
    inv_ref[...] = jnp.zeros((_NET * _N, _C), jnp.float32)

    def body(a, _):
        frow = flat_ref[a]
        for j in range(128):
            f = frow[j]
            inv_ref[f, :] = inv_ref[f, :] + 1.0
        return _

    lax.fori_loop(0, _EROWS, body, None)
    inv_ref[...] = 1.0 / jnp.maximum(inv_ref[...], 1.0)


def _conv_body(col_ref, flat_ref, h_ref, inv_ref, w_ref, o_ref, sums_ref):
    t = pl.program_id(0)

    @pl.when(t == 0)
    def _scatter():
        sums_ref[...] = jnp.zeros((_NET * _N, _C), jnp.float32)

        def body(a, _):
            crow = col_ref[a]
            frow = flat_ref[a]
            for j in range(128):
                c = crow[j]
                f = frow[j]
                sums_ref[f, :] = sums_ref[f, :] + h_ref[c, :]
            return _

        lax.fori_loop(0, _EROWS, body, None)
        o_ref[...] = jnp.zeros((_N, _C), jnp.float32)

    mean = sums_ref[pl.ds(t * _N, _N), :] * inv_ref[...]
    o_ref[...] += jnp.dot(mean, w_ref[...],
                          preferred_element_type=jnp.float32)


def _make_conv():
    full = lambda t: (0, 0)
    in_specs = [
        pl.BlockSpec((_EROWS, 128), full),
        pl.BlockSpec((_EROWS, 128), full),
        pl.BlockSpec((_N, _C), full),
        pl.BlockSpec((_N, _C), lambda t: (t, 0)),
        pl.BlockSpec((_C, _C), lambda t: (t, 0)),
    ]
    return pl.pallas_call(
        _conv_body,
        grid=(_NET,),
        in_specs=in_specs,
        out_specs=pl.BlockSpec((_N, _C), full),
        out_shape=jax.ShapeDtypeStruct((_N, _C), jnp.float32),
        scratch_shapes=[pltpu.VMEM((_NET * _N, _C), jnp.float32)],
    )


def kernel(x, emb, edge_index, edge_dir, batch_id, gn1_w, gn1_b, W1, emb_w,
           emb_b, gn2_w, gn2_b, W2):
    row, col = edge_index[0], edge_index[1]
    flat = edge_dir * _N + row
    col2d = col.reshape(_EROWS, 128)
    flat2d = flat.reshape(_EROWS, 128)
    bid_col = batch_id.reshape(_N, 1)
    gmat = _group_mat()

    gn1 = pl.pallas_call(
        _gn1_body,
        out_shape=jax.ShapeDtypeStruct((_N, _C), jnp.float32),
    )
    h = gn1(x, bid_col, gn1_w, gn1_b, gmat)

    inv_cnt = pl.pallas_call(
        _counts_body,
        out_shape=jax.ShapeDtypeStruct((_NET * _N, _C), jnp.float32),
    )(flat2d)

    conv = _make_conv()
    h = conv(col2d, flat2d, h, inv_cnt, W1)

    gn2 = pl.pallas_call(
        _gn2_body,
        out_shape=jax.ShapeDtypeStruct((_N, _C), jnp.float32),
    )
    h = gn2(h, emb, emb_w, emb_b.reshape(1, _C), bid_col, gn2_w, gn2_b, gmat)

    return x + conv(col2d, flat2d, h, inv_cnt, W2)
